# baseline (speedup 1.0000x reference)
import dataclasses
import functools

import jax
import jax.numpy as jnp
from jax import lax
from jax.experimental import pallas as pl
from jax.experimental.pallas import tpu as pltpu
from jax.experimental.pallas import tpu_sc as plsc

F = 128
H = 4
C = 32
DE = F + 16
AT = 16
NW = 32
K = 96
ZR = 16


def _build_prep(np_, bp):
    def body(x_ref, w_ref, a16_ref, xle_ref, atab_ref):
        xl = jnp.dot(x_ref[...], w_ref[...], preferred_element_type=jnp.float32)
        cols = lax.broadcasted_iota(jnp.int32, (bp, 16), 1)
        extra = jnp.where(cols < H, 1.0, 0.0).astype(jnp.float32)
        xle_ref[...] = jnp.concatenate([xl, extra], axis=1)
        atab_ref[...] = jnp.dot(xl, a16_ref[...], preferred_element_type=jnp.float32)

    return pl.pallas_call(
        body,
        grid=(np_ // bp,),
        in_specs=[
            pl.BlockSpec((bp, F), lambda i: (i, 0)),
            pl.BlockSpec((F, F), lambda i: (0, 0)),
            pl.BlockSpec((F, AT), lambda i: (0, 0)),
        ],
        out_specs=[
            pl.BlockSpec((bp, DE), lambda i: (i, 0)),
            pl.BlockSpec((bp, AT), lambda i: (i, 0)),
        ],
        out_shape=[
            jax.ShapeDtypeStruct((np_, DE), jnp.float32),
            jax.ShapeDtypeStruct((np_, AT), jnp.float32),
        ],
    )


def _build_sc(np_, epw):
    iters = epw // K
    assert iters % 2 == 0
    rows_per_sub = np_ // 16
    mesh = plsc.VectorSubcoreMesh(core_axis_name="c", subcore_axis_name="s")
    cp = pltpu.CompilerParams()
    if "needs_layout_passes" in pltpu.CompilerParams.__dataclass_fields__:
        cp = dataclasses.replace(cp, needs_layout_passes=False)
    if "use_tc_tiling_on_sc" in pltpu.CompilerParams.__dataclass_fields__:
        cp = dataclasses.replace(cp, use_tc_tiling_on_sc=False)

    @functools.partial(
        pl.kernel,
        compiler_params=cp,
        out_type=jax.ShapeDtypeStruct((2, np_, DE), jnp.float32),
        mesh=mesh,
        scratch_types=[
            pltpu.VMEM((K, DE), jnp.float32),
            pltpu.VMEM((K, DE), jnp.float32),
            pltpu.VMEM((K, AT), jnp.float32),
            pltpu.VMEM((K, AT), jnp.float32),
            pltpu.VMEM((K, AT), jnp.float32),
            pltpu.VMEM((K, AT), jnp.float32),
            pltpu.VMEM((K * H,), jnp.float32),
            pltpu.VMEM((K,), jnp.int32),
            pltpu.VMEM((K,), jnp.int32),
            pltpu.VMEM((K,), jnp.int32),
            pltpu.VMEM((K,), jnp.int32),
            pltpu.VMEM((ZR, DE), jnp.float32),
            pltpu.VMEM_SHARED((np_, DE), jnp.float32),
            pltpu.SemaphoreType.DMA,
            pltpu.SemaphoreType.DMA,
            pltpu.SemaphoreType.DMA,
            pltpu.SemaphoreType.DMA,
            pltpu.SemaphoreType.DMA,
            pltpu.SemaphoreType.DMA,
            pltpu.SemaphoreType.DMA,
            pltpu.SemaphoreType.DMA,
            pltpu.SemaphoreType.DMA,
            pltpu.SemaphoreType.DMA,
        ],
    )
    def sc_gat(xle_hbm, atab_hbm, src_hbm, dst_hbm, out_hbm,
               rows0_v, rows1_v, asr0_v, asr1_v, adr0_v, adr1_v, ea_v,
               src0_v, src1_v, dst0_v, dst1_v, zbuf_v, acc_sh,
               sr0, sr1, sa0, sa1, sb0, sb1, ss0, ss1, sd0, sd1):
        rows_b = (rows0_v, rows1_v)
        asr_b = (asr0_v, asr1_v)
        adr_b = (adr0_v, adr1_v)
        src_b = (src0_v, src1_v)
        dst_b = (dst0_v, dst1_v)
        s_rows = (sr0, sr1)
        s_asr = (sa0, sa1)
        s_adr = (sb0, sb1)
        s_src = (ss0, ss1)
        s_dst = (sd0, sd1)
        c = lax.axis_index("c")
        s = lax.axis_index("s")
        wid = c * 16 + s
        iota16 = lax.iota(jnp.int32, 16)

        @pl.loop(0, ZR)
        def _(i):
            for j in range(DE // 16):
                zbuf_v[i, pl.ds(16 * j, 16)] = jnp.zeros((16,), jnp.float32)

        @pl.loop(0, rows_per_sub // ZR)
        def _(t):
            pltpu.sync_copy(zbuf_v, acc_sh.at[pl.ds(s * rows_per_sub + t * ZR, ZR)])

        plsc.subcore_barrier()

        def start_idx(chunk, b):
            base = wid * epw + chunk * K
            pltpu.async_copy(src_hbm.at[pl.ds(base, K)], src_b[b], s_src[b])
            pltpu.async_copy(dst_hbm.at[pl.ds(base, K)], dst_b[b], s_dst[b])

        def wait_idx(b):
            pltpu.make_async_copy(src_hbm.at[pl.ds(0, K)], src_b[b], s_src[b]).wait()
            pltpu.make_async_copy(dst_hbm.at[pl.ds(0, K)], dst_b[b], s_dst[b]).wait()

        def start_gather(b):
            pltpu.async_copy(xle_hbm.at[src_b[b]], rows_b[b], s_rows[b])
            pltpu.async_copy(atab_hbm.at[src_b[b]], asr_b[b], s_asr[b])
            pltpu.async_copy(atab_hbm.at[dst_b[b]], adr_b[b], s_adr[b])

        def wait_gather(b):
            pltpu.make_async_copy(xle_hbm.at[src_b[b]], rows_b[b], s_rows[b]).wait()
            pltpu.make_async_copy(atab_hbm.at[src_b[b]], asr_b[b], s_asr[b]).wait()
            pltpu.make_async_copy(atab_hbm.at[dst_b[b]], adr_b[b], s_adr[b]).wait()

        base0 = wid * epw
        pltpu.sync_copy(src_hbm.at[pl.ds(base0, K)], src_b[0])
        pltpu.sync_copy(dst_hbm.at[pl.ds(base0, K)], dst_b[0])
        start_gather(0)
        start_idx(1, 1)

        @pl.loop(0, iters // 2)
        def _(g):
            for b in (0, 1):
                it = 2 * g + b
                o = 1 - b
                wait_idx(o)
                start_gather(o)
                wait_gather(b)

                rv = rows_b[b]
                for gg in range(K // 16):
                    ev = iota16 + 16 * gg
                    for h in range(H):
                        a = (plsc.load_gather(asr_b[b], [ev, jnp.full((16,), h, jnp.int32)])
                             + plsc.load_gather(adr_b[b], [ev, jnp.full((16,), H + h, jnp.int32)]))
                        a = jnp.maximum(a, 0.2 * a)
                        plsc.store_scatter(ea_v, [ev * H + h], jnp.exp(a))

                @pl.loop(0, K, step=4)
                def _(e0):
                    for q in range(4):
                        e = e0 + q
                        eb = H * e
                        for h in range(H):
                            bb = plsc.load_gather(
                                ea_v, [jnp.full((16,), eb + h, jnp.int32)])
                            for jj in (2 * h, 2 * h + 1):
                                rv[e, pl.ds(16 * jj, 16)] = (
                                    rv[e, pl.ds(16 * jj, 16)] * bb)
                        bb = plsc.load_gather(ea_v, [eb + (iota16 & (H - 1))])
                        rv[e, pl.ds(F, 16)] = rv[e, pl.ds(F, 16)] * bb

                pltpu.sync_copy(rows_b[b], acc_sh.at[dst_b[b]], add=True)
                start_idx(jnp.minimum(it + 2, iters - 1), b)

        wait_gather(0)
        wait_idx(1)

        plsc.subcore_barrier()
        pltpu.sync_copy(acc_sh.at[pl.ds(s * rows_per_sub, rows_per_sub)],
                        out_hbm.at[c, pl.ds(s * rows_per_sub, rows_per_sub)])

    return sc_gat


def _build_fin(np_, bf):
    def body(p_ref, x_ref, b_ref, o_ref):
        sall = p_ref[0] + p_ref[1]
        acc = sall[:, :F]
        outs = []
        for h in range(H):
            ah = sall[:, F + h:F + h + 1]
            outs.append(acc[:, C * h:C * (h + 1)] / (ah + 1e-16))
        o = jnp.concatenate(outs, axis=1) + b_ref[...]
        o = jnp.where(o > 0, o, jnp.exp(o) - 1.0)
        o_ref[...] = o + x_ref[...]

    return pl.pallas_call(
        body,
        grid=(np_ // bf,),
        in_specs=[
            pl.BlockSpec((2, bf, DE), lambda i: (0, i, 0)),
            pl.BlockSpec((bf, F), lambda i: (i, 0)),
            pl.BlockSpec((1, F), lambda i: (0, 0)),
        ],
        out_specs=pl.BlockSpec((bf, F), lambda i: (i, 0)),
        out_shape=jax.ShapeDtypeStruct((np_, F), jnp.float32),
    )


def kernel(x, edge_index, W, att_src, att_dst, bias):
    n = x.shape[0]
    e = edge_index.shape[1]
    np_ = ((n + 1 + 1023) // 1024) * 1024
    etot = e + n
    epw = ((etot + NW * 2 * K - 1) // (NW * 2 * K)) * 2 * K
    ep = NW * epw

    ei = edge_index.astype(jnp.int32)
    loop = jnp.arange(n, dtype=jnp.int32)
    padi = jnp.full((ep - etot,), n, jnp.int32)
    src = jnp.concatenate([ei[0], loop, padi])
    dst = jnp.concatenate([ei[1], loop, padi])
    xp = jnp.zeros((np_, F), jnp.float32).at[:n, :].set(x)

    asr = att_src.reshape(H, C)
    adr = att_dst.reshape(H, C)
    eyeh = jnp.eye(H, dtype=jnp.float32)
    a16 = jnp.concatenate(
        [(asr[:, :, None] * eyeh[:, None, :]).reshape(H * C, H),
         (adr[:, :, None] * eyeh[:, None, :]).reshape(H * C, H),
         jnp.zeros((H * C, AT - 2 * H), jnp.float32)], axis=1)

    xle, atab = _build_prep(np_, 2048)(xp, W, a16)
    part = _build_sc(np_, epw)(xle, atab, src, dst)
    outp = _build_fin(np_, 1024)(part, xp, bias.reshape(1, F))
    return outp[:n]


# speedup vs baseline: 1.0602x; 1.0602x over previous
import dataclasses
import functools

import jax
import jax.numpy as jnp
from jax import lax
from jax.experimental import pallas as pl
from jax.experimental.pallas import tpu as pltpu
from jax.experimental.pallas import tpu_sc as plsc

F = 128
H = 4
C = 32
DE = F + 16
AT = 16
NW = 32
K = 96
ZR = 16


def _build_prep(np_, bp):
    def body(x_ref, w_ref, a16_ref, xle_ref, atab_ref):
        xl = jnp.dot(x_ref[...], w_ref[...], preferred_element_type=jnp.float32)
        cols = lax.broadcasted_iota(jnp.int32, (bp, 16), 1)
        extra = jnp.where(cols < H, 1.0, 0.0).astype(jnp.float32)
        xle_ref[...] = jnp.concatenate([xl, extra], axis=1)
        atab_ref[...] = jnp.dot(xl, a16_ref[...], preferred_element_type=jnp.float32)

    return pl.pallas_call(
        body,
        grid=(np_ // bp,),
        in_specs=[
            pl.BlockSpec((bp, F), lambda i: (i, 0)),
            pl.BlockSpec((F, F), lambda i: (0, 0)),
            pl.BlockSpec((F, AT), lambda i: (0, 0)),
        ],
        out_specs=[
            pl.BlockSpec((bp, DE), lambda i: (i, 0)),
            pl.BlockSpec((bp, AT), lambda i: (i, 0)),
        ],
        out_shape=[
            jax.ShapeDtypeStruct((np_, DE), jnp.float32),
            jax.ShapeDtypeStruct((np_, AT), jnp.float32),
        ],
    )


def _build_sc(np_, epw):
    iters = epw // K
    assert iters % 2 == 0
    rows_per_sub = np_ // 16
    mesh = plsc.VectorSubcoreMesh(core_axis_name="c", subcore_axis_name="s")
    cp = pltpu.CompilerParams()
    if "needs_layout_passes" in pltpu.CompilerParams.__dataclass_fields__:
        cp = dataclasses.replace(cp, needs_layout_passes=False)
    if "use_tc_tiling_on_sc" in pltpu.CompilerParams.__dataclass_fields__:
        cp = dataclasses.replace(cp, use_tc_tiling_on_sc=False)

    @functools.partial(
        pl.kernel,
        compiler_params=cp,
        out_type=jax.ShapeDtypeStruct((2, np_, DE), jnp.float32),
        mesh=mesh,
        scratch_types=[
            pltpu.VMEM((K, DE), jnp.float32),
            pltpu.VMEM((K, DE), jnp.float32),
            pltpu.VMEM((K, AT), jnp.float32),
            pltpu.VMEM((K, AT), jnp.float32),
            pltpu.VMEM((K, AT), jnp.float32),
            pltpu.VMEM((K, AT), jnp.float32),
            pltpu.VMEM((K * H,), jnp.float32),
            pltpu.VMEM((K,), jnp.int32),
            pltpu.VMEM((K,), jnp.int32),
            pltpu.VMEM((K,), jnp.int32),
            pltpu.VMEM((K,), jnp.int32),
            pltpu.VMEM((K,), jnp.int32),
            pltpu.VMEM((K,), jnp.int32),
            pltpu.VMEM((ZR, DE), jnp.float32),
            pltpu.VMEM_SHARED((np_, DE), jnp.float32),
            pltpu.SemaphoreType.DMA,
            pltpu.SemaphoreType.DMA,
            pltpu.SemaphoreType.DMA,
            pltpu.SemaphoreType.DMA,
            pltpu.SemaphoreType.DMA,
            pltpu.SemaphoreType.DMA,
            pltpu.SemaphoreType.DMA,
            pltpu.SemaphoreType.DMA,
            pltpu.SemaphoreType.DMA,
            pltpu.SemaphoreType.DMA,
            pltpu.SemaphoreType.DMA,
            pltpu.SemaphoreType.DMA,
        ],
    )
    def sc_gat(xle_hbm, atab_hbm, src_hbm, dst_hbm, out_hbm,
               rows0_v, rows1_v, asr0_v, asr1_v, adr0_v, adr1_v, ea_v,
               src0_v, src1_v, dst0_v, dst1_v, scd0_v, scd1_v, zbuf_v, acc_sh,
               sr0, sr1, sa0, sa1, sb0, sb1, ss0, ss1, sd0, sd1, sc0, sc1):
        rows_b = (rows0_v, rows1_v)
        asr_b = (asr0_v, asr1_v)
        adr_b = (adr0_v, adr1_v)
        src_b = (src0_v, src1_v)
        dst_b = (dst0_v, dst1_v)
        scd_b = (scd0_v, scd1_v)
        s_rows = (sr0, sr1)
        s_asr = (sa0, sa1)
        s_adr = (sb0, sb1)
        s_src = (ss0, ss1)
        s_dst = (sd0, sd1)
        s_sct = (sc0, sc1)
        c = lax.axis_index("c")
        s = lax.axis_index("s")
        wid = c * 16 + s
        iota16 = lax.iota(jnp.int32, 16)

        @pl.loop(0, ZR)
        def _(i):
            for j in range(DE // 16):
                zbuf_v[i, pl.ds(16 * j, 16)] = jnp.zeros((16,), jnp.float32)

        @pl.loop(0, rows_per_sub // ZR)
        def _(t):
            pltpu.sync_copy(zbuf_v, acc_sh.at[pl.ds(s * rows_per_sub + t * ZR, ZR)])

        plsc.subcore_barrier()

        def start_idx(chunk, b):
            base = wid * epw + chunk * K
            pltpu.async_copy(src_hbm.at[pl.ds(base, K)], src_b[b], s_src[b])
            pltpu.async_copy(dst_hbm.at[pl.ds(base, K)], dst_b[b], s_dst[b])

        def wait_idx(b):
            pltpu.make_async_copy(src_hbm.at[pl.ds(0, K)], src_b[b], s_src[b]).wait()
            pltpu.make_async_copy(dst_hbm.at[pl.ds(0, K)], dst_b[b], s_dst[b]).wait()

        def start_gather(b):
            pltpu.async_copy(xle_hbm.at[src_b[b]], rows_b[b], s_rows[b])
            pltpu.async_copy(atab_hbm.at[src_b[b]], asr_b[b], s_asr[b])
            pltpu.async_copy(atab_hbm.at[dst_b[b]], adr_b[b], s_adr[b])

        def wait_gather(b):
            pltpu.make_async_copy(xle_hbm.at[src_b[b]], rows_b[b], s_rows[b]).wait()
            pltpu.make_async_copy(atab_hbm.at[src_b[b]], asr_b[b], s_asr[b]).wait()
            pltpu.make_async_copy(atab_hbm.at[dst_b[b]], adr_b[b], s_adr[b]).wait()

        def start_scatter(b):
            pltpu.async_copy(rows_b[b], acc_sh.at[scd_b[b]], s_sct[b], add=True)

        def wait_scatter(b):
            pltpu.make_async_copy(rows_b[b], acc_sh.at[scd_b[b]], s_sct[b]).wait()

        @pl.loop(0, K)
        def _(r):
            for j in range(DE // 16):
                rows_b[1][r, pl.ds(16 * j, 16)] = jnp.zeros((16,), jnp.float32)

        @pl.loop(0, K // 16)
        def _(t):
            scd_b[1][pl.ds(16 * t, 16)] = jnp.zeros((16,), jnp.int32)

        start_scatter(1)

        base0 = wid * epw
        pltpu.sync_copy(src_hbm.at[pl.ds(base0, K)], src_b[0])
        pltpu.sync_copy(dst_hbm.at[pl.ds(base0, K)], dst_b[0])
        start_gather(0)
        start_idx(1, 1)

        @pl.loop(0, iters // 2)
        def _(g):
            for b in (0, 1):
                it = 2 * g + b
                o = 1 - b
                wait_idx(o)
                wait_scatter(o)
                start_gather(o)
                wait_gather(b)

                rv = rows_b[b]
                for gg in range(K // 16):
                    ev = iota16 + 16 * gg
                    for h in range(H):
                        a = (plsc.load_gather(asr_b[b], [ev, jnp.full((16,), h, jnp.int32)])
                             + plsc.load_gather(adr_b[b], [ev, jnp.full((16,), H + h, jnp.int32)]))
                        a = jnp.maximum(a, 0.2 * a)
                        plsc.store_scatter(ea_v, [ev * H + h], jnp.exp(a))

                @pl.loop(0, K, step=4)
                def _(e0):
                    for q in range(4):
                        e = e0 + q
                        eb = H * e
                        for h in range(H):
                            bb = plsc.load_gather(
                                ea_v, [jnp.full((16,), eb + h, jnp.int32)])
                            for jj in (2 * h, 2 * h + 1):
                                rv[e, pl.ds(16 * jj, 16)] = (
                                    rv[e, pl.ds(16 * jj, 16)] * bb)
                        bb = plsc.load_gather(ea_v, [eb + (iota16 & (H - 1))])
                        rv[e, pl.ds(F, 16)] = rv[e, pl.ds(F, 16)] * bb

                @pl.loop(0, K // 16)
                def _(t):
                    scd_b[b][pl.ds(16 * t, 16)] = dst_b[b][pl.ds(16 * t, 16)]

                start_scatter(b)
                start_idx(jnp.minimum(it + 2, iters - 1), b)

        wait_gather(0)
        wait_idx(1)
        wait_scatter(1)

        plsc.subcore_barrier()
        pltpu.sync_copy(acc_sh.at[pl.ds(s * rows_per_sub, rows_per_sub)],
                        out_hbm.at[c, pl.ds(s * rows_per_sub, rows_per_sub)])

    return sc_gat


def _build_fin(np_, bf):
    def body(p_ref, x_ref, b_ref, o_ref):
        sall = p_ref[0] + p_ref[1]
        acc = sall[:, :F]
        outs = []
        for h in range(H):
            ah = sall[:, F + h:F + h + 1]
            outs.append(acc[:, C * h:C * (h + 1)] / (ah + 1e-16))
        o = jnp.concatenate(outs, axis=1) + b_ref[...]
        o = jnp.where(o > 0, o, jnp.exp(o) - 1.0)
        o_ref[...] = o + x_ref[...]

    return pl.pallas_call(
        body,
        grid=(np_ // bf,),
        in_specs=[
            pl.BlockSpec((2, bf, DE), lambda i: (0, i, 0)),
            pl.BlockSpec((bf, F), lambda i: (i, 0)),
            pl.BlockSpec((1, F), lambda i: (0, 0)),
        ],
        out_specs=pl.BlockSpec((bf, F), lambda i: (i, 0)),
        out_shape=jax.ShapeDtypeStruct((np_, F), jnp.float32),
    )


def kernel(x, edge_index, W, att_src, att_dst, bias):
    n = x.shape[0]
    e = edge_index.shape[1]
    np_ = ((n + 1 + 1023) // 1024) * 1024
    etot = e + n
    epw = ((etot + NW * 2 * K - 1) // (NW * 2 * K)) * 2 * K
    ep = NW * epw

    ei = edge_index.astype(jnp.int32)
    loop = jnp.arange(n, dtype=jnp.int32)
    padi = jnp.full((ep - etot,), n, jnp.int32)
    src = jnp.concatenate([ei[0], loop, padi])
    dst = jnp.concatenate([ei[1], loop, padi])
    xp = jnp.zeros((np_, F), jnp.float32).at[:n, :].set(x)

    asr = att_src.reshape(H, C)
    adr = att_dst.reshape(H, C)
    eyeh = jnp.eye(H, dtype=jnp.float32)
    a16 = jnp.concatenate(
        [(asr[:, :, None] * eyeh[:, None, :]).reshape(H * C, H),
         (adr[:, :, None] * eyeh[:, None, :]).reshape(H * C, H),
         jnp.zeros((H * C, AT - 2 * H), jnp.float32)], axis=1)

    xle, atab = _build_prep(np_, 2048)(xp, W, a16)
    part = _build_sc(np_, epw)(xle, atab, src, dst)
    outp = _build_fin(np_, 1024)(part, xp, bias.reshape(1, F))
    return outp[:n]
